# baseline (speedup 1.0000x reference)
import functools
import jax
import jax.numpy as jnp
from jax.experimental import pallas as pl
from jax.experimental.pallas import tpu as pltpu

T = 2048
D = 1024
F = 2048
E = 8
TOPK = 2
BT = 256


def _moe_body(x_ref, g_ref, w1_ref, w3_ref, w2_ref, o_ref):
    e = pl.program_id(1)
    x = x_ref[...]

    logits = jnp.dot(x, g_ref[...], preferred_element_type=jnp.float32)
    m = jnp.max(logits, axis=1, keepdims=True)
    ex = jnp.exp(logits - m)
    p = ex / jnp.sum(ex, axis=1, keepdims=True)

    idx = jax.lax.broadcasted_iota(jnp.int32, (BT, E), 1)
    w1v = jnp.max(p, axis=1, keepdims=True)
    i1 = jnp.min(jnp.where(p == w1v, idx, E), axis=1, keepdims=True)
    p2 = jnp.where(idx == i1, -1.0, p)
    w2v = jnp.max(p2, axis=1, keepdims=True)
    i2 = jnp.min(jnp.where(p2 == w2v, idx, E), axis=1, keepdims=True)
    denom = w1v + w2v
    ew = jnp.where(i1 == e, w1v, jnp.where(i2 == e, w2v, 0.0)) / denom

    a = jnp.dot(x, w1_ref[0], preferred_element_type=jnp.float32)
    b = jnp.dot(x, w3_ref[0], preferred_element_type=jnp.float32)
    h = (a * jax.lax.logistic(a)) * b
    y = jnp.dot(h, w2_ref[0], preferred_element_type=jnp.float32)

    @pl.when(e == 0)
    def _():
        o_ref[...] = jnp.zeros_like(o_ref)

    o_ref[...] += ew * y


@jax.jit
def kernel(hidden_states, gate_w, w1, w2, w3):
    grid = (T // BT, E)
    return pl.pallas_call(
        _moe_body,
        grid=grid,
        in_specs=[
            pl.BlockSpec((BT, D), lambda t, e: (t, 0)),
            pl.BlockSpec((D, E), lambda t, e: (0, 0)),
            pl.BlockSpec((1, D, F), lambda t, e: (e, 0, 0)),
            pl.BlockSpec((1, D, F), lambda t, e: (e, 0, 0)),
            pl.BlockSpec((1, F, D), lambda t, e: (e, 0, 0)),
        ],
        out_specs=pl.BlockSpec((BT, D), lambda t, e: (t, 0)),
        out_shape=jax.ShapeDtypeStruct((T, D), jnp.float32),
        compiler_params=pltpu.CompilerParams(
            dimension_semantics=("parallel", "arbitrary"),
        ),
    )(hidden_states, gate_w, w1, w3, w2)


# speedup vs baseline: 1.7301x; 1.7301x over previous
import jax
import jax.numpy as jnp
from jax.experimental import pallas as pl
from jax.experimental.pallas import tpu as pltpu

T = 2048
D = 1024
F = 2048
E = 8
TOPK = 2
BT = 512
BF = 1024


def _moe_body(x_ref, g_ref, w1_ref, w3_ref, w2_ref, o_ref):
    e = pl.program_id(0)
    f = pl.program_id(1)
    t = pl.program_id(2)
    x = x_ref[pl.ds(t * BT, BT), :]

    logits = jnp.dot(x, g_ref[...], preferred_element_type=jnp.float32)
    m = jnp.max(logits, axis=1, keepdims=True)
    ex = jnp.exp(logits - m)
    p = ex / jnp.sum(ex, axis=1, keepdims=True)

    idx = jax.lax.broadcasted_iota(jnp.int32, (BT, E), 1)
    w1v = jnp.max(p, axis=1, keepdims=True)
    i1 = jnp.min(jnp.where(p == w1v, idx, E), axis=1, keepdims=True)
    p2 = jnp.where(idx == i1, -1.0, p)
    w2v = jnp.max(p2, axis=1, keepdims=True)
    i2 = jnp.min(jnp.where(p2 == w2v, idx, E), axis=1, keepdims=True)
    denom = w1v + w2v
    ew = jnp.where(i1 == e, w1v, jnp.where(i2 == e, w2v, 0.0)) / denom

    a = jnp.dot(x, w1_ref[0], preferred_element_type=jnp.float32)
    b = jnp.dot(x, w3_ref[0], preferred_element_type=jnp.float32)
    h = (a * jax.lax.logistic(a)) * b
    y = jnp.dot(h, w2_ref[0], preferred_element_type=jnp.float32)

    @pl.when((e == 0) & (f == 0))
    def _():
        o_ref[pl.ds(t * BT, BT), :] = jnp.zeros((BT, D), jnp.float32)

    o_ref[pl.ds(t * BT, BT), :] += ew * y


@jax.jit
def kernel(hidden_states, gate_w, w1, w2, w3):
    grid = (E, F // BF, T // BT)
    return pl.pallas_call(
        _moe_body,
        grid=grid,
        in_specs=[
            pl.BlockSpec((T, D), lambda e, f, t: (0, 0)),
            pl.BlockSpec((D, E), lambda e, f, t: (0, 0)),
            pl.BlockSpec((1, D, BF), lambda e, f, t: (e, 0, f)),
            pl.BlockSpec((1, D, BF), lambda e, f, t: (e, 0, f)),
            pl.BlockSpec((1, BF, D), lambda e, f, t: (e, f, 0)),
        ],
        out_specs=pl.BlockSpec((T, D), lambda e, f, t: (0, 0)),
        out_shape=jax.ShapeDtypeStruct((T, D), jnp.float32),
        compiler_params=pltpu.CompilerParams(
            dimension_semantics=("arbitrary", "arbitrary", "arbitrary"),
        ),
    )(hidden_states, gate_w, w1, w3, w2)
